# baseline (speedup 1.0000x reference)
import dataclasses
import functools

import jax
import jax.numpy as jnp
from jax import lax
from jax.experimental import pallas as pl
from jax.experimental.pallas import tpu as pltpu
from jax.experimental.pallas import tpu_sc as plsc

N = 100000
E = 3200000
G = 256
BLK = 1024
NPAD = 100352
ROWS = NPAD // 16
NW = 32
EPW = E // NW
EB = 2000
NB = EPW // EB

_MESH = plsc.VectorSubcoreMesh(core_axis_name="c", subcore_axis_name="s")


def _strip_space(x):
    from jax._src import core as _jcore
    from jax._src.pallas import core as _pl_core
    return _pl_core.with_memory_space_constraint_p.bind(
        x, memory_space=_jcore.MemorySpace.Device)

_SC_PARAMS = pltpu.CompilerParams()
if "needs_layout_passes" in pltpu.CompilerParams.__dataclass_fields__:
    _SC_PARAMS = dataclasses.replace(
        _SC_PARAMS, needs_layout_passes=False, use_tc_tiling_on_sc=False)


def _deg_body(dst_hbm, out_hbm, idx0, idx1, deg2d, sem0, sem1):
    c = lax.axis_index("c")
    s = lax.axis_index("s")
    wid = c * 16 + s
    base = wid * EPW

    @pl.loop(0, ROWS)
    def _(i):
        deg2d[i, :] = jnp.zeros((16,), jnp.float32)

    ones = jnp.ones((16,), jnp.float32)

    def start(buf, sem, b):
        pltpu.async_copy(dst_hbm.at[pl.ds(base + b * EB, EB)], buf, sem)

    def wait(buf, sem):
        pltpu.make_async_copy(dst_hbm.at[pl.ds(base, EB)], buf, sem).wait()

    def process(buf):
        @pl.loop(0, EB, step=16)
        def _(j):
            d = buf[pl.ds(j, 16)]
            row = lax.shift_right_logical(d, 4)
            col = jnp.bitwise_and(d, 15)
            plsc.addupdate_scatter(deg2d, [row, col], ones)

    start(idx0, sem0, 0)

    @pl.loop(0, NB // 2)
    def _(p):
        b = p * 2
        start(idx1, sem1, b + 1)
        wait(idx0, sem0)
        process(idx0)

        @pl.when(p < NB // 2 - 1)
        def _():
            start(idx0, sem0, b + 2)

        wait(idx1, sem1)
        process(idx1)

    pltpu.sync_copy(deg2d, out_hbm.at[wid])


@jax.jit
def _sc_degree(dst):
    k = pl.kernel(
        _deg_body,
        out_type=pltpu.HBM((NW, ROWS, 16), jnp.float32),
        mesh=_MESH,
        compiler_params=_SC_PARAMS,
        scratch_types=[
            pltpu.VMEM((EB,), jnp.int32),
            pltpu.VMEM((EB,), jnp.int32),
            pltpu.VMEM((ROWS, 16), jnp.float32),
            pltpu.SemaphoreType.DMA,
            pltpu.SemaphoreType.DMA,
        ],
    )
    return k(dst)


EPAD = 3276800
SB = 1024
SROW = SB // 128
NSB = EPAD // 16 // SB
HN = NPAD // 2
ACC_ROWS = HN + 256
ZB = ACC_ROWS // 16
ZBLK = ZB // 16
IROWS = EPAD // 128


def _make_edge_agg_body(passes):
    def body(tab_hbm, src_hbm, dst_hbm, out_hbm,
             sbuf0, sbuf1, dbuf0, dbuf1, rows0, rows1, zrows, shared,
             semi0, semi1, semg0, semg1, sems0, sems1):
        c = lax.axis_index("c")
        s = lax.axis_index("s")
        row_base = s * (NSB * SB)

        sbufs = (sbuf0, sbuf1)
        dbufs = (dbuf0, dbuf1)
        rowss = (rows0, rows1)
        semis = (semi0, semi1)
        semgs = (semg0, semg1)
        semss = (sems0, sems1)

        @pl.loop(0, ZBLK)
        def _(i):
            zrows[i, :] = jnp.zeros((16,), jnp.float32)

        def zero_acc():
            @pl.loop(0, 16)
            def _(i):
                pltpu.sync_copy(zrows,
                                shared.at[pl.ds(s * ZB + i * ZBLK, ZBLK)])

        for q in range(passes):
            dplane = c if passes == 1 else q

            def start_idx(b, p):
                pltpu.async_copy(
                    src_hbm.at[c, pl.ds(row_base + b * SB, SB)],
                    sbufs[p], semis[p])
                pltpu.async_copy(
                    dst_hbm.at[dplane, pl.ds(row_base + b * SB, SB)],
                    dbufs[p], semis[p])

            def wait_idx(p):
                pltpu.make_async_copy(src_hbm.at[c, pl.ds(0, SB)],
                                      sbufs[p], semis[p]).wait()
                pltpu.make_async_copy(dst_hbm.at[dplane, pl.ds(0, SB)],
                                      dbufs[p], semis[p]).wait()

            def fire_gathers(p):
                pltpu.async_copy(tab_hbm.at[sbufs[p]], rowss[p],
                                 semgs[p]).wait()

            def fire_scatters(p):
                pltpu.async_copy(rowss[p], shared.at[dbufs[p]], semss[p],
                                 add=True)

            def drain_scatters(p):
                pltpu.make_async_copy(rowss[p], shared.at[dbufs[p]],
                                      semss[p]).wait()

            zero_acc()
            plsc.subcore_barrier()
            start_idx(0, 0)

            @pl.loop(0, NSB // 2)
            def _(p):
                b0 = p * 2
                wait_idx(0)
                fire_gathers(0)

                @pl.when(p > 0)
                def _():
                    drain_scatters(1)

                fire_scatters(0)
                start_idx(b0 + 1, 1)
                wait_idx(1)
                fire_gathers(1)
                drain_scatters(0)
                fire_scatters(1)

                @pl.when(p < NSB // 2 - 1)
                def _():
                    start_idx(b0 + 2, 0)

            drain_scatters(1)
            plsc.subcore_barrier()
            if passes == 1:
                dst_out = out_hbm.at[c, pl.ds(s * (HN // 16), HN // 16)]
            else:
                dst_out = out_hbm.at[c, q, pl.ds(s * (HN // 16), HN // 16)]
            pltpu.sync_copy(shared.at[pl.ds(s * (HN // 16), HN // 16)],
                            dst_out)
            if q + 1 < passes:
                plsc.subcore_barrier()

    return body


def _edge_agg_call(body, out_shape, tab, src_arr, dst_arr):
    k = pl.kernel(
        body,
        out_type=pltpu.HBM(out_shape, jnp.float32),
        mesh=_MESH,
        compiler_params=_SC_PARAMS,
        scratch_types=[
            pltpu.VMEM((SB,), jnp.int32),
            pltpu.VMEM((SB,), jnp.int32),
            pltpu.VMEM((SB,), jnp.int32),
            pltpu.VMEM((SB,), jnp.int32),
            pltpu.VMEM((SB, 16), jnp.float32),
            pltpu.VMEM((SB, 16), jnp.float32),
            pltpu.VMEM((ZBLK, 16), jnp.float32),
            pltpu.VMEM_SHARED((ACC_ROWS, 16), jnp.float32),
            pltpu.SemaphoreType.DMA,
            pltpu.SemaphoreType.DMA,
            pltpu.SemaphoreType.DMA,
            pltpu.SemaphoreType.DMA,
            pltpu.SemaphoreType.DMA,
            pltpu.SemaphoreType.DMA,
        ],
    )
    return _strip_space(k(tab, src_arr, dst_arr))


@jax.jit
def _sc_agg_b(g1t, src2b, dst2n):
    body = _make_edge_agg_body(passes=1)
    return _edge_agg_call(body, (2, HN, 16), g1t, src2b, dst2n)


@jax.jit
def _sc_agg_c(g2flat, src2c, dst2n):
    body = _make_edge_agg_body(passes=2)
    return _edge_agg_call(body, (2, 2, HN, 16), g2flat, src2c, dst2n)


def _m1_body(aggb_ref, g1t_ref, dinv_ref, w1_ref, b1_ref, w2_ref, g2_ref):
    dinv = dinv_ref[...]
    z1 = dinv * (aggb_ref[...][0] + g1t_ref[...])
    h1 = jnp.maximum(
        jax.lax.dot(z1, w1_ref[...], preferred_element_type=jnp.float32)
        + b1_ref[...], 0.0)
    m2 = jax.lax.dot(h1, w2_ref[...], preferred_element_type=jnp.float32)
    g2_ref[...] = dinv * m2


def _m1_call(aggb, g1t, dinv2d, W1p, b1, W2):
    return pl.pallas_call(
        _m1_body,
        grid=(NPAD // BLK,),
        in_specs=[
            pl.BlockSpec((1, BLK, 16), lambda i: (i // 49, i % 49, 0)),
            pl.BlockSpec((BLK, 16), lambda i: (i, 0)),
            pl.BlockSpec((BLK, 1), lambda i: (i, 0)),
            pl.BlockSpec((16, 64), lambda i: (0, 0)),
            pl.BlockSpec((1, 64), lambda i: (0, 0)),
            pl.BlockSpec((64, 32), lambda i: (0, 0)),
        ],
        out_specs=pl.BlockSpec((BLK, 32), lambda i: (i, 0)),
        out_shape=jax.ShapeDtypeStruct((NPAD, 32), jnp.float32),
    )(aggb, g1t, dinv2d, W1p, b1.reshape(1, 64), W2)


def _pool_body(batch_ref, agg2_ref, g2_ref, dinv_ref, b2_ref, w3_ref,
               b3_ref, out_ref, acc_ref, cnt_ref):
    step = pl.program_id(0)

    @pl.when(step == 0)
    def _():
        acc_ref[...] = jnp.zeros_like(acc_ref)
        cnt_ref[...] = jnp.zeros_like(cnt_ref)

    h2 = jnp.maximum(
        dinv_ref[...] * (agg2_ref[...] + g2_ref[...]) + b2_ref[...], 0.0)
    ids = batch_ref[...][0]
    onehot = (ids == lax.broadcasted_iota(jnp.int32, (G, BLK), 0)).astype(
        jnp.float32
    )
    acc_ref[...] += lax.dot_general(
        onehot, h2, (((1,), (0,)), ((), ())),
        preferred_element_type=jnp.float32,
    )
    cnt_ref[...] += jnp.sum(onehot, axis=1, keepdims=True)

    @pl.when(step == pl.num_programs(0) - 1)
    def _():
        pooled = acc_ref[...] / jnp.maximum(cnt_ref[...], 1.0)
        out_ref[...] = (
            jax.lax.dot(pooled, w3_ref[...],
                        preferred_element_type=jnp.float32) + b3_ref[...])


def _pool_call(agg2, g2, dinv2d, batch, b2, W3, b3):
    bpad = jnp.pad(batch.astype(jnp.int32), (0, NPAD - N), constant_values=G)
    b3d = bpad.reshape(NPAD // BLK, 1, BLK)
    return pl.pallas_call(
        _pool_body,
        grid=(NPAD // BLK,),
        in_specs=[
            pl.BlockSpec((1, 1, BLK), lambda i: (i, 0, 0)),
            pl.BlockSpec((BLK, 32), lambda i: (i, 0)),
            pl.BlockSpec((BLK, 32), lambda i: (i, 0)),
            pl.BlockSpec((BLK, 1), lambda i: (i, 0)),
            pl.BlockSpec((1, 32), lambda i: (0, 0)),
            pl.BlockSpec((32, 5), lambda i: (0, 0)),
            pl.BlockSpec((1, 5), lambda i: (0, 0)),
        ],
        out_specs=pl.BlockSpec((G, 5), lambda i: (0, 0)),
        out_shape=jax.ShapeDtypeStruct((G, 5), jnp.float32),
        scratch_shapes=[pltpu.VMEM((G, 32), jnp.float32),
                        pltpu.VMEM((G, 1), jnp.float32)],
    )(b3d, agg2, g2, dinv2d, b2.reshape(1, 32), W3, b3.reshape(1, 5))


def kernel(x, edge_index, batch, W1, b1, W2, b2, W3, b3):
    edge_index = edge_index.astype(jnp.int32)
    batch = batch.astype(jnp.int32)
    src, dst = edge_index[0], edge_index[1]

    degp = _strip_space(_sc_degree(dst))
    deg = 1.0 + degp.sum(axis=0).reshape(NPAD)
    dinv = lax.rsqrt(deg)
    dinv2d = dinv.reshape(NPAD, 1)

    srcp = jnp.concatenate([src, jnp.zeros((EPAD - E,), jnp.int32)])
    dstp = jnp.concatenate([dst, jnp.full((EPAD - E,), NPAD, jnp.int32)])
    dst_lo = jnp.where(dstp < HN, dstp, HN)
    dst_hi = jnp.where((dstp >= HN) & (dstp < NPAD), dstp - HN, HN)
    dst2n = jnp.stack([dst_lo, dst_hi])
    src2b = jnp.stack([srcp, srcp])
    src2c = jnp.stack([srcp, srcp + NPAD])

    g1t = jnp.zeros((NPAD, 16), jnp.float32)
    g1t = g1t.at[:N, :7].set(dinv[:N, None] * x)
    aggb = _sc_agg_b(g1t, src2b, dst2n)

    W1p = jnp.zeros((16, 64), jnp.float32).at[:7].set(W1)
    g2 = _m1_call(aggb, g1t, dinv2d, W1p, b1, W2)

    g2p = jnp.stack([g2[:, :16], g2[:, 16:]])
    aggc = _sc_agg_c(g2p.reshape(2 * NPAD, 16), src2c, dst2n)
    agg2 = jnp.concatenate(
        [jnp.concatenate([aggc[0, 0], aggc[1, 0]], axis=1),
         jnp.concatenate([aggc[0, 1], aggc[1, 1]], axis=1)], axis=0)

    return _pool_call(agg2, g2, dinv2d, batch, b2, W3, b3)


# speedup vs baseline: 1.0001x; 1.0001x over previous
import dataclasses
import functools

import jax
import jax.numpy as jnp
from jax import lax
from jax.experimental import pallas as pl
from jax.experimental.pallas import tpu as pltpu
from jax.experimental.pallas import tpu_sc as plsc

N = 100000
E = 3200000
G = 256
BLK = 1024
NPAD = 100352
ROWS = NPAD // 16
NW = 32
EPW = E // NW
EB = 2000
NB = EPW // EB

_MESH = plsc.VectorSubcoreMesh(core_axis_name="c", subcore_axis_name="s")


def _strip_space(x):
    from jax._src import core as _jcore
    from jax._src.pallas import core as _pl_core
    return _pl_core.with_memory_space_constraint_p.bind(
        x, memory_space=_jcore.MemorySpace.Device)

_SC_PARAMS = pltpu.CompilerParams()
if "needs_layout_passes" in pltpu.CompilerParams.__dataclass_fields__:
    _SC_PARAMS = dataclasses.replace(
        _SC_PARAMS, needs_layout_passes=False, use_tc_tiling_on_sc=False)


def _deg_body(dst_hbm, out_hbm, idx0, idx1, deg2d, sem0, sem1):
    c = lax.axis_index("c")
    s = lax.axis_index("s")
    wid = c * 16 + s
    base = wid * EPW

    @pl.loop(0, ROWS)
    def _(i):
        deg2d[i, :] = jnp.zeros((16,), jnp.float32)

    ones = jnp.ones((16,), jnp.float32)

    def start(buf, sem, b):
        pltpu.async_copy(dst_hbm.at[pl.ds(base + b * EB, EB)], buf, sem)

    def wait(buf, sem):
        pltpu.make_async_copy(dst_hbm.at[pl.ds(base, EB)], buf, sem).wait()

    def process(buf):
        @pl.loop(0, EB, step=16)
        def _(j):
            d = buf[pl.ds(j, 16)]
            row = lax.shift_right_logical(d, 4)
            col = jnp.bitwise_and(d, 15)
            plsc.addupdate_scatter(deg2d, [row, col], ones)

    start(idx0, sem0, 0)

    @pl.loop(0, NB // 2)
    def _(p):
        b = p * 2
        start(idx1, sem1, b + 1)
        wait(idx0, sem0)
        process(idx0)

        @pl.when(p < NB // 2 - 1)
        def _():
            start(idx0, sem0, b + 2)

        wait(idx1, sem1)
        process(idx1)

    pltpu.sync_copy(deg2d, out_hbm.at[wid])


@jax.jit
def _sc_degree(dst):
    k = pl.kernel(
        _deg_body,
        out_type=pltpu.HBM((NW, ROWS, 16), jnp.float32),
        mesh=_MESH,
        compiler_params=_SC_PARAMS,
        scratch_types=[
            pltpu.VMEM((EB,), jnp.int32),
            pltpu.VMEM((EB,), jnp.int32),
            pltpu.VMEM((ROWS, 16), jnp.float32),
            pltpu.SemaphoreType.DMA,
            pltpu.SemaphoreType.DMA,
        ],
    )
    return k(dst)


EPAD = 3276800
SB = 1024
SROW = SB // 128
NSB = EPAD // 16 // SB
HN = NPAD // 2
ACC_ROWS = HN + 256
ZB = ACC_ROWS // 16
ZBLK = ZB // 16
IROWS = EPAD // 128


def _make_edge_agg_body(passes):
    def body(tab_hbm, src_hbm, dst_hbm, zer_hbm, out_hbm,
             sbuf0, sbuf1, dbuf0, dbuf1, rows0, rows1, shared,
             semi0, semi1, semg0, semg1, sems0, sems1):
        c = lax.axis_index("c")
        s = lax.axis_index("s")
        row_base = s * (NSB * SB)

        sbufs = (sbuf0, sbuf1)
        dbufs = (dbuf0, dbuf1)
        rowss = (rows0, rows1)
        semis = (semi0, semi1)
        semgs = (semg0, semg1)
        semss = (sems0, sems1)

        def zero_acc():
            pltpu.sync_copy(zer_hbm, shared.at[pl.ds(s * ZB, ZB)])

        for q in range(passes):
            dplane = c if passes == 1 else q

            def start_idx(b, p):
                pltpu.async_copy(
                    src_hbm.at[c, pl.ds(row_base + b * SB, SB)],
                    sbufs[p], semis[p])
                pltpu.async_copy(
                    dst_hbm.at[dplane, pl.ds(row_base + b * SB, SB)],
                    dbufs[p], semis[p])

            def wait_idx(p):
                pltpu.make_async_copy(src_hbm.at[c, pl.ds(0, SB)],
                                      sbufs[p], semis[p]).wait()
                pltpu.make_async_copy(dst_hbm.at[dplane, pl.ds(0, SB)],
                                      dbufs[p], semis[p]).wait()

            def fire_gathers(p):
                pltpu.async_copy(tab_hbm.at[sbufs[p]], rowss[p],
                                 semgs[p]).wait()

            def fire_scatters(p):
                pltpu.async_copy(rowss[p], shared.at[dbufs[p]], semss[p],
                                 add=True)

            def drain_scatters(p):
                pltpu.make_async_copy(rowss[p], shared.at[dbufs[p]],
                                      semss[p]).wait()

            zero_acc()
            plsc.subcore_barrier()
            start_idx(0, 0)

            @pl.loop(0, NSB // 2)
            def _(p):
                b0 = p * 2
                wait_idx(0)
                fire_gathers(0)

                @pl.when(p > 0)
                def _():
                    drain_scatters(1)

                fire_scatters(0)
                start_idx(b0 + 1, 1)
                wait_idx(1)
                fire_gathers(1)
                drain_scatters(0)
                fire_scatters(1)

                @pl.when(p < NSB // 2 - 1)
                def _():
                    start_idx(b0 + 2, 0)

            drain_scatters(1)
            plsc.subcore_barrier()
            if passes == 1:
                dst_out = out_hbm.at[c, pl.ds(s * (HN // 16), HN // 16)]
            else:
                dst_out = out_hbm.at[c, q, pl.ds(s * (HN // 16), HN // 16)]
            pltpu.sync_copy(shared.at[pl.ds(s * (HN // 16), HN // 16)],
                            dst_out)
            if q + 1 < passes:
                plsc.subcore_barrier()

    return body


def _edge_agg_call(body, out_shape, width, tab, src_arr, dst_arr, zer):
    k = pl.kernel(
        body,
        out_type=pltpu.HBM(out_shape, jnp.float32),
        mesh=_MESH,
        compiler_params=_SC_PARAMS,
        scratch_types=[
            pltpu.VMEM((SB,), jnp.int32),
            pltpu.VMEM((SB,), jnp.int32),
            pltpu.VMEM((SB,), jnp.int32),
            pltpu.VMEM((SB,), jnp.int32),
            pltpu.VMEM((SB, width), jnp.float32),
            pltpu.VMEM((SB, width), jnp.float32),
            pltpu.VMEM_SHARED((ACC_ROWS, width), jnp.float32),
            pltpu.SemaphoreType.DMA,
            pltpu.SemaphoreType.DMA,
            pltpu.SemaphoreType.DMA,
            pltpu.SemaphoreType.DMA,
            pltpu.SemaphoreType.DMA,
            pltpu.SemaphoreType.DMA,
        ],
    )
    return _strip_space(k(tab, src_arr, dst_arr, zer))


@jax.jit
def _sc_agg_b(g1t, src2b, dst2n, zer8):
    body = _make_edge_agg_body(passes=1)
    return _edge_agg_call(body, (2, HN, 8), 8, g1t, src2b, dst2n, zer8)


@jax.jit
def _sc_agg_c(g2flat, src2c, dst2n, zer16):
    body = _make_edge_agg_body(passes=2)
    return _edge_agg_call(body, (2, 2, HN, 16), 16, g2flat, src2c, dst2n,
                          zer16)


def _m1_body(aggb_ref, g1t_ref, dinv_ref, w1_ref, b1_ref, w2_ref, g2_ref):
    dinv = dinv_ref[...]
    z1 = dinv * (aggb_ref[...][0] + g1t_ref[...])
    h1 = jnp.maximum(
        jax.lax.dot(z1, w1_ref[...], preferred_element_type=jnp.float32)
        + b1_ref[...], 0.0)
    m2 = jax.lax.dot(h1, w2_ref[...], preferred_element_type=jnp.float32)
    g2_ref[...] = dinv * m2


def _m1_call(aggb, g1t, dinv2d, W1p, b1, W2):
    return pl.pallas_call(
        _m1_body,
        grid=(NPAD // BLK,),
        in_specs=[
            pl.BlockSpec((1, BLK, 8), lambda i: (i // 49, i % 49, 0)),
            pl.BlockSpec((BLK, 8), lambda i: (i, 0)),
            pl.BlockSpec((BLK, 1), lambda i: (i, 0)),
            pl.BlockSpec((8, 64), lambda i: (0, 0)),
            pl.BlockSpec((1, 64), lambda i: (0, 0)),
            pl.BlockSpec((64, 32), lambda i: (0, 0)),
        ],
        out_specs=pl.BlockSpec((BLK, 32), lambda i: (i, 0)),
        out_shape=jax.ShapeDtypeStruct((NPAD, 32), jnp.float32),
    )(aggb, g1t, dinv2d, W1p, b1.reshape(1, 64), W2)


def _pool_body(batch_ref, agg2_ref, g2_ref, dinv_ref, b2_ref, w3_ref,
               b3_ref, out_ref, acc_ref, cnt_ref):
    step = pl.program_id(0)

    @pl.when(step == 0)
    def _():
        acc_ref[...] = jnp.zeros_like(acc_ref)
        cnt_ref[...] = jnp.zeros_like(cnt_ref)

    h2 = jnp.maximum(
        dinv_ref[...] * (agg2_ref[...] + g2_ref[...]) + b2_ref[...], 0.0)
    ids = batch_ref[...][0]
    onehot = (ids == lax.broadcasted_iota(jnp.int32, (G, BLK), 0)).astype(
        jnp.float32
    )
    acc_ref[...] += lax.dot_general(
        onehot, h2, (((1,), (0,)), ((), ())),
        preferred_element_type=jnp.float32,
    )
    cnt_ref[...] += jnp.sum(onehot, axis=1, keepdims=True)

    @pl.when(step == pl.num_programs(0) - 1)
    def _():
        pooled = acc_ref[...] / jnp.maximum(cnt_ref[...], 1.0)
        out_ref[...] = (
            jax.lax.dot(pooled, w3_ref[...],
                        preferred_element_type=jnp.float32) + b3_ref[...])


def _pool_call(agg2, g2, dinv2d, batch, b2, W3, b3):
    bpad = jnp.pad(batch.astype(jnp.int32), (0, NPAD - N), constant_values=G)
    b3d = bpad.reshape(NPAD // BLK, 1, BLK)
    return pl.pallas_call(
        _pool_body,
        grid=(NPAD // BLK,),
        in_specs=[
            pl.BlockSpec((1, 1, BLK), lambda i: (i, 0, 0)),
            pl.BlockSpec((BLK, 32), lambda i: (i, 0)),
            pl.BlockSpec((BLK, 32), lambda i: (i, 0)),
            pl.BlockSpec((BLK, 1), lambda i: (i, 0)),
            pl.BlockSpec((1, 32), lambda i: (0, 0)),
            pl.BlockSpec((32, 5), lambda i: (0, 0)),
            pl.BlockSpec((1, 5), lambda i: (0, 0)),
        ],
        out_specs=pl.BlockSpec((G, 5), lambda i: (0, 0)),
        out_shape=jax.ShapeDtypeStruct((G, 5), jnp.float32),
        scratch_shapes=[pltpu.VMEM((G, 32), jnp.float32),
                        pltpu.VMEM((G, 1), jnp.float32)],
    )(b3d, agg2, g2, dinv2d, b2.reshape(1, 32), W3, b3.reshape(1, 5))


def kernel(x, edge_index, batch, W1, b1, W2, b2, W3, b3):
    edge_index = edge_index.astype(jnp.int32)
    batch = batch.astype(jnp.int32)
    src, dst = edge_index[0], edge_index[1]

    degp = _strip_space(_sc_degree(dst))
    deg = 1.0 + degp.sum(axis=0).reshape(NPAD)
    dinv = lax.rsqrt(deg)
    dinv2d = dinv.reshape(NPAD, 1)

    srcp = jnp.concatenate([src, jnp.zeros((EPAD - E,), jnp.int32)])
    dstp = jnp.concatenate([dst, jnp.full((EPAD - E,), NPAD, jnp.int32)])
    dst_lo = jnp.where(dstp < HN, dstp, HN)
    dst_hi = jnp.where((dstp >= HN) & (dstp < NPAD), dstp - HN, HN)
    dst2n = jnp.stack([dst_lo, dst_hi])
    src2b = jnp.stack([srcp, srcp])
    src2c = jnp.stack([srcp, srcp + NPAD])

    g1t = jnp.zeros((NPAD, 8), jnp.float32)
    g1t = g1t.at[:N, :7].set(dinv[:N, None] * x)
    zer8 = jnp.zeros((ZB, 8), jnp.float32)
    zer16 = jnp.zeros((ZB, 16), jnp.float32)
    aggb = _sc_agg_b(g1t, src2b, dst2n, zer8)

    W1p = jnp.zeros((8, 64), jnp.float32).at[:7].set(W1)
    g2 = _m1_call(aggb, g1t, dinv2d, W1p, b1, W2)

    g2p = jnp.stack([g2[:, :16], g2[:, 16:]])
    aggc = _sc_agg_c(g2p.reshape(2 * NPAD, 16), src2c, dst2n,
                     zer16)
    agg2 = jnp.concatenate(
        [jnp.concatenate([aggc[0, 0], aggc[1, 0]], axis=1),
         jnp.concatenate([aggc[0, 1], aggc[1, 1]], axis=1)], axis=0)

    return _pool_call(agg2, g2, dinv2d, batch, b2, W3, b3)


# speedup vs baseline: 2.7510x; 2.7506x over previous
import dataclasses
import functools

import jax
import jax.numpy as jnp
from jax import lax
from jax.experimental import pallas as pl
from jax.experimental.pallas import tpu as pltpu
from jax.experimental.pallas import tpu_sc as plsc

N = 100000
E = 3200000
G = 256
BLK = 1024
NPAD = 100352
ROWS = NPAD // 16
NW = 32
EPW = E // NW
EB = 2000
NB = EPW // EB

_MESH = plsc.VectorSubcoreMesh(core_axis_name="c", subcore_axis_name="s")


def _strip_space(x):
    from jax._src import core as _jcore
    from jax._src.pallas import core as _pl_core
    return _pl_core.with_memory_space_constraint_p.bind(
        x, memory_space=_jcore.MemorySpace.Device)

_SC_PARAMS = pltpu.CompilerParams()
if "needs_layout_passes" in pltpu.CompilerParams.__dataclass_fields__:
    _SC_PARAMS = dataclasses.replace(
        _SC_PARAMS, needs_layout_passes=False, use_tc_tiling_on_sc=False)


def _deg_body(dst_hbm, out_hbm, idx0, idx1, deg2d, sem0, sem1):
    c = lax.axis_index("c")
    s = lax.axis_index("s")
    wid = c * 16 + s
    base = wid * EPW

    @pl.loop(0, ROWS)
    def _(i):
        deg2d[i, :] = jnp.zeros((16,), jnp.float32)

    ones = jnp.ones((16,), jnp.float32)

    def start(buf, sem, b):
        pltpu.async_copy(dst_hbm.at[pl.ds(base + b * EB, EB)], buf, sem)

    def wait(buf, sem):
        pltpu.make_async_copy(dst_hbm.at[pl.ds(base, EB)], buf, sem).wait()

    def process(buf):
        @pl.loop(0, EB, step=16)
        def _(j):
            d = buf[pl.ds(j, 16)]
            row = lax.shift_right_logical(d, 4)
            col = jnp.bitwise_and(d, 15)
            plsc.addupdate_scatter(deg2d, [row, col], ones)

    start(idx0, sem0, 0)

    @pl.loop(0, NB // 2)
    def _(p):
        b = p * 2
        start(idx1, sem1, b + 1)
        wait(idx0, sem0)
        process(idx0)

        @pl.when(p < NB // 2 - 1)
        def _():
            start(idx0, sem0, b + 2)

        wait(idx1, sem1)
        process(idx1)

    pltpu.sync_copy(deg2d, out_hbm.at[wid])


@jax.jit
def _sc_degree(dst):
    k = pl.kernel(
        _deg_body,
        out_type=pltpu.HBM((NW, ROWS, 16), jnp.float32),
        mesh=_MESH,
        compiler_params=_SC_PARAMS,
        scratch_types=[
            pltpu.VMEM((EB,), jnp.int32),
            pltpu.VMEM((EB,), jnp.int32),
            pltpu.VMEM((ROWS, 16), jnp.float32),
            pltpu.SemaphoreType.DMA,
            pltpu.SemaphoreType.DMA,
        ],
    )
    return k(dst)


EPAD = 3276800
ACCF = NPAD + 16
ZBF = ACCF // 16


def _make_edge_agg_body(nsb, sb, edge_split):
    def body(tab_hbm, src_hbm, dst_hbm, zer_hbm, out_hbm,
             sbuf0, sbuf1, dbuf0, dbuf1, rows0, rows1, shared,
             semi0, semi1, semg0, semg1, sems0, sems1):
        c = lax.axis_index("c")
        s = lax.axis_index("s")
        if edge_split:
            row_base = (c * 16 + s) * (nsb * sb)
        else:
            row_base = s * (nsb * sb)

        sbufs = (sbuf0, sbuf1)
        dbufs = (dbuf0, dbuf1)
        rowss = (rows0, rows1)
        semis = (semi0, semi1)
        semgs = (semg0, semg1)
        semss = (sems0, sems1)

        def start_idx(b, p):
            pltpu.async_copy(
                src_hbm.at[c, pl.ds(row_base + b * sb, sb)],
                sbufs[p], semis[p])
            pltpu.async_copy(
                dst_hbm.at[0, pl.ds(row_base + b * sb, sb)],
                dbufs[p], semis[p])

        def wait_idx(p):
            pltpu.make_async_copy(src_hbm.at[c, pl.ds(0, sb)],
                                  sbufs[p], semis[p]).wait()
            pltpu.make_async_copy(dst_hbm.at[0, pl.ds(0, sb)],
                                  dbufs[p], semis[p]).wait()

        def fire_gathers(p):
            pltpu.async_copy(tab_hbm.at[sbufs[p]], rowss[p],
                             semgs[p]).wait()

        def fire_scatters(p):
            pltpu.async_copy(rowss[p], shared.at[dbufs[p]], semss[p],
                             add=True)

        def drain_scatters(p):
            pltpu.make_async_copy(rowss[p], shared.at[dbufs[p]],
                                  semss[p]).wait()

        pltpu.sync_copy(zer_hbm, shared.at[pl.ds(s * ZBF, ZBF)])
        plsc.subcore_barrier()
        start_idx(0, 0)

        @pl.loop(0, nsb // 2)
        def _(p):
            b0 = p * 2
            wait_idx(0)
            fire_gathers(0)

            @pl.when(p > 0)
            def _():
                drain_scatters(1)

            fire_scatters(0)
            start_idx(b0 + 1, 1)
            wait_idx(1)
            fire_gathers(1)
            drain_scatters(0)
            fire_scatters(1)

            @pl.when(p < nsb // 2 - 1)
            def _():
                start_idx(b0 + 2, 0)

        drain_scatters(1)
        plsc.subcore_barrier()
        pltpu.sync_copy(shared.at[pl.ds(s * (NPAD // 16), NPAD // 16)],
                        out_hbm.at[c, pl.ds(s * (NPAD // 16), NPAD // 16)])

    return body


def _edge_agg_call(body, out_shape, width, sb, tab, src_arr, dst_arr, zer):
    k = pl.kernel(
        body,
        out_type=pltpu.HBM(out_shape, jnp.float32),
        mesh=_MESH,
        compiler_params=_SC_PARAMS,
        scratch_types=[
            pltpu.VMEM((sb,), jnp.int32),
            pltpu.VMEM((sb,), jnp.int32),
            pltpu.VMEM((sb,), jnp.int32),
            pltpu.VMEM((sb,), jnp.int32),
            pltpu.VMEM((sb, width), jnp.float32),
            pltpu.VMEM((sb, width), jnp.float32),
            pltpu.VMEM_SHARED((ACCF, width), jnp.float32),
            pltpu.SemaphoreType.DMA,
            pltpu.SemaphoreType.DMA,
            pltpu.SemaphoreType.DMA,
            pltpu.SemaphoreType.DMA,
            pltpu.SemaphoreType.DMA,
            pltpu.SemaphoreType.DMA,
        ],
    )
    return _strip_space(k(tab, src_arr, dst_arr, zer))


@jax.jit
def _sc_agg_b(g1t, src2b, dstg, zer8):
    sb = 1024
    body = _make_edge_agg_body(EPAD // 32 // sb, sb, edge_split=True)
    return _edge_agg_call(body, (2, NPAD, 8), 8, sb, g1t, src2b, dstg, zer8)


@jax.jit
def _sc_agg_c(g2flat, src2c, dstg, zer16):
    sb = 512
    body = _make_edge_agg_body(EPAD // 16 // sb, sb, edge_split=False)
    return _edge_agg_call(body, (2, NPAD, 16), 16, sb, g2flat, src2c, dstg,
                          zer16)


def _m1_body(aggb_ref, g1t_ref, dinv_ref, w1_ref, b1_ref, w2_ref, g2_ref):
    dinv = dinv_ref[...]
    z1 = dinv * (aggb_ref[...] + g1t_ref[...])
    h1 = jnp.maximum(
        jax.lax.dot(z1, w1_ref[...], preferred_element_type=jnp.float32)
        + b1_ref[...], 0.0)
    m2 = jax.lax.dot(h1, w2_ref[...], preferred_element_type=jnp.float32)
    g2_ref[...] = dinv * m2


def _m1_call(aggb, g1t, dinv2d, W1p, b1, W2):
    return pl.pallas_call(
        _m1_body,
        grid=(NPAD // BLK,),
        in_specs=[
            pl.BlockSpec((BLK, 8), lambda i: (i, 0)),
            pl.BlockSpec((BLK, 8), lambda i: (i, 0)),
            pl.BlockSpec((BLK, 1), lambda i: (i, 0)),
            pl.BlockSpec((8, 64), lambda i: (0, 0)),
            pl.BlockSpec((1, 64), lambda i: (0, 0)),
            pl.BlockSpec((64, 32), lambda i: (0, 0)),
        ],
        out_specs=pl.BlockSpec((BLK, 32), lambda i: (i, 0)),
        out_shape=jax.ShapeDtypeStruct((NPAD, 32), jnp.float32),
    )(aggb, g1t, dinv2d, W1p, b1.reshape(1, 64), W2)


def _pool_body(batch_ref, agg2_ref, g2_ref, dinv_ref, b2_ref, w3_ref,
               b3_ref, out_ref, acc_ref, cnt_ref):
    step = pl.program_id(0)

    @pl.when(step == 0)
    def _():
        acc_ref[...] = jnp.zeros_like(acc_ref)
        cnt_ref[...] = jnp.zeros_like(cnt_ref)

    h2 = jnp.maximum(
        dinv_ref[...] * (agg2_ref[...] + g2_ref[...]) + b2_ref[...], 0.0)
    ids = batch_ref[...][0]
    onehot = (ids == lax.broadcasted_iota(jnp.int32, (G, BLK), 0)).astype(
        jnp.float32
    )
    acc_ref[...] += lax.dot_general(
        onehot, h2, (((1,), (0,)), ((), ())),
        preferred_element_type=jnp.float32,
    )
    cnt_ref[...] += jnp.sum(onehot, axis=1, keepdims=True)

    @pl.when(step == pl.num_programs(0) - 1)
    def _():
        pooled = acc_ref[...] / jnp.maximum(cnt_ref[...], 1.0)
        out_ref[...] = (
            jax.lax.dot(pooled, w3_ref[...],
                        preferred_element_type=jnp.float32) + b3_ref[...])


def _pool_call(agg2, g2, dinv2d, batch, b2, W3, b3):
    bpad = jnp.pad(batch.astype(jnp.int32), (0, NPAD - N), constant_values=G)
    b3d = bpad.reshape(NPAD // BLK, 1, BLK)
    return pl.pallas_call(
        _pool_body,
        grid=(NPAD // BLK,),
        in_specs=[
            pl.BlockSpec((1, 1, BLK), lambda i: (i, 0, 0)),
            pl.BlockSpec((BLK, 32), lambda i: (i, 0)),
            pl.BlockSpec((BLK, 32), lambda i: (i, 0)),
            pl.BlockSpec((BLK, 1), lambda i: (i, 0)),
            pl.BlockSpec((1, 32), lambda i: (0, 0)),
            pl.BlockSpec((32, 5), lambda i: (0, 0)),
            pl.BlockSpec((1, 5), lambda i: (0, 0)),
        ],
        out_specs=pl.BlockSpec((G, 5), lambda i: (0, 0)),
        out_shape=jax.ShapeDtypeStruct((G, 5), jnp.float32),
        scratch_shapes=[pltpu.VMEM((G, 32), jnp.float32),
                        pltpu.VMEM((G, 1), jnp.float32)],
    )(b3d, agg2, g2, dinv2d, b2.reshape(1, 32), W3, b3.reshape(1, 5))


def kernel(x, edge_index, batch, W1, b1, W2, b2, W3, b3):
    edge_index = edge_index.astype(jnp.int32)
    batch = batch.astype(jnp.int32)
    src, dst = edge_index[0], edge_index[1]

    degp = _strip_space(_sc_degree(dst))
    deg = 1.0 + degp.sum(axis=0).reshape(NPAD)
    dinv = lax.rsqrt(deg)
    dinv2d = dinv.reshape(NPAD, 1)

    srcp = jnp.concatenate([src, jnp.zeros((EPAD - E,), jnp.int32)])
    dstp = jnp.concatenate([dst, jnp.full((EPAD - E,), NPAD, jnp.int32)])
    dstg = dstp.reshape(1, EPAD)
    src2b = jnp.stack([srcp, srcp])
    src2c = jnp.stack([srcp, srcp + NPAD])

    g1t = jnp.zeros((NPAD, 8), jnp.float32)
    g1t = g1t.at[:N, :7].set(dinv[:N, None] * x)
    zer8 = jnp.zeros((ZBF, 8), jnp.float32)
    zer16 = jnp.zeros((ZBF, 16), jnp.float32)
    aggb = _sc_agg_b(g1t, src2b, dstg, zer8)
    aggb1 = aggb[0] + aggb[1]

    W1p = jnp.zeros((8, 64), jnp.float32).at[:7].set(W1)
    g2 = _m1_call(aggb1, g1t, dinv2d, W1p, b1, W2)

    g2p = jnp.stack([g2[:, :16], g2[:, 16:]])
    aggc = _sc_agg_c(g2p.reshape(2 * NPAD, 16), src2c, dstg,
                     zer16)
    agg2 = jnp.concatenate([aggc[0], aggc[1]], axis=1)

    return _pool_call(agg2, g2, dinv2d, batch, b2, W3, b3)
